# baseline (speedup 1.0000x reference)
import functools

import jax
import jax.numpy as jnp
from jax.experimental import pallas as pl
from jax.experimental.pallas import tpu as pltpu

B, T, EMB = 2, 2048, 768
HEADS, DEPTH = 12, 2
BUCKET, NHASH, FF_CHUNKS = 64, 4, 16
D = EMB // HEADS
BN = T // BUCKET
CHUNKS = BN * NHASH


def _layernorm(x, g, b):
    mu = jnp.mean(x, axis=-1, keepdims=True)
    var = jnp.mean((x - mu) ** 2, axis=-1, keepdims=True)
    return g * (x - mu) / jnp.sqrt(var + 1e-3) + b


def _ffn(x, W1, b1, W2, b2):
    return jnp.maximum(x @ W1 + b1, 0.0) @ W2 + b2


def _chunk_ffn(x, g, be, W1, b1, W2, b2):
    h = _layernorm(x, g, be)
    h = _ffn(h, W1, b1, W2, b2)
    chunks = jnp.split(h, FF_CHUNKS, axis=-2)
    return jnp.concatenate([_ffn(c, W1, b1, W2, b2) for c in chunks], axis=-2)


def _look_forward(x):
    xf = jnp.concatenate([x[:, -1:], x[:, :-1]], axis=1)
    return jnp.concatenate([x, xf], axis=2)


def _lsh_attention(qk, v, key):
    Bh, S, Dh = qk.shape
    bucket_nums = S // BUCKET
    chunknum = bucket_nums * NHASH
    r_size = bucket_nums // 2
    R = jax.random.normal(key, (Bh, Dh, NHASH, r_size), dtype=jnp.float32)
    xR = jnp.einsum('btf,bfhi->bhti', qk, R)
    xR = jnp.concatenate([xR, -xR], axis=-1)
    hash_off = (bucket_nums * jnp.arange(NHASH)).reshape(1, -1, 1)
    buckets = jnp.argmax(xR, axis=-1) + hash_off
    buckets = buckets.reshape(Bh, -1)
    bucket_index = jnp.arange(NHASH * S)
    scaled = S * buckets + (bucket_index % S)[None, :]
    arg_sort = jnp.argsort(scaled, axis=-1)
    undo = jnp.argsort(arg_sort, axis=-1)
    h_idx = arg_sort % S
    sorted_qk = jnp.take_along_axis(qk, h_idx[..., None], axis=1).reshape(Bh, chunknum, -1, Dh)
    sorted_v = jnp.take_along_axis(v, h_idx[..., None], axis=1).reshape(Bh, chunknum, -1, Dh)
    sq_idx = h_idx.reshape(Bh, chunknum, -1)
    sorted_q = sorted_qk
    sorted_k = sorted_qk / jnp.linalg.norm(sorted_qk, axis=-1, keepdims=True)
    sorted_k = _look_forward(sorted_k)
    sorted_v = _look_forward(sorted_v)
    skv_idx = _look_forward(sq_idx)
    attn = jnp.einsum('bhie,bhje->bhij', sorted_q, sorted_k) * (float(Dh) ** -0.5)
    self_mask = (sq_idx[:, :, :, None] == skv_idx[:, :, None, :]).astype(jnp.float32)
    attn = attn * (1.0 - self_mask) + self_mask * (-1e5)
    lse = jax.scipy.special.logsumexp(attn, axis=-1, keepdims=True)
    attn = jnp.exp(attn - lse)
    sorted_qkv = jnp.einsum('buij,buje->buie', attn, sorted_v).reshape(Bh, -1, Dh)
    sorted_logits = lse.reshape(Bh, -1)
    qkv = jnp.take_along_axis(sorted_qkv, undo[..., None], axis=1)
    logits = jnp.take_along_axis(sorted_logits, undo, axis=-1)
    qkv = qkv.reshape(Bh, NHASH, S, Dh)
    logits = logits.reshape(Bh, NHASH, S, 1)
    ratio = jnp.exp(logits - jax.scipy.special.logsumexp(logits, axis=1, keepdims=True))
    return jnp.sum(qkv * ratio, axis=1)


def _mh_lsh(x, Wk, Wv, Wo, bo, key):
    b, t, e = x.shape
    h = HEADS
    qk = x @ Wk
    v = x @ Wv
    def split_heads(z):
        return jnp.transpose(z.reshape(b, t, h, -1), (0, 2, 1, 3))
    qkh = split_heads(qk).reshape(b * h, t, -1)
    vh = split_heads(v).reshape(b * h, t, -1)
    outs = []
    for i in range(h):
        outs.append(_lsh_attention(qkh[i * b:(i + 1) * b], vh[i * b:(i + 1) * b], jax.random.fold_in(key, i)))
    attn_out = jnp.concatenate(outs, axis=0)
    out = jnp.transpose(attn_out.reshape(b, t, h, -1), (0, 2, 1, 3)).reshape(b, t, e)
    return out @ Wo + bo


def kernel(x, Wk0, Wv0, Wo0, bo0, g0, be0, W1_0, b1_0, W2_0, b2_0, Wk1, Wv1, Wo1, bo1, g1, be1, W1_1, b1_1, W2_1, b2_1):
    params = [
        (Wk0, Wv0, Wo0, bo0, g0, be0, W1_0, b1_0, W2_0, b2_0),
        (Wk1, Wv1, Wo1, bo1, g1, be1, W1_1, b1_1, W2_1, b2_1),
    ]
    key = jax.random.key(42)
    hcat = jnp.concatenate([x, x], axis=-1)
    for d, (Wk, Wv, Wo, bo, g, be, W1, b1, W2, b2) in enumerate(params):
        x1, x2 = jnp.split(hcat, 2, axis=-1)
        y1 = _mh_lsh(x2, Wk, Wv, Wo, bo, jax.random.fold_in(key, d)) + x1
        y2 = _chunk_ffn(y1, g, be, W1, b1, W2, b2) + x2
        hcat = jnp.concatenate([y1, y2], axis=-1)
    return hcat
